# baseline (speedup 1.0000x reference)
import functools

import jax
import jax.numpy as jnp
from jax import lax
from jax.experimental import pallas as pl
from jax.experimental.pallas import tpu as pltpu
from jax.experimental.pallas import tpu_sc as plsc

_ALPHA = 1.0


def _blend_kernel(image_ref, outlier_ref, mask_ref, image_out_ref):
    m = mask_ref[...]
    img = image_ref[...]
    out = outlier_ref[...]
    blended = img - _ALPHA * m[:, None] * img + _ALPHA * out
    image_out_ref[...] = blended.astype(jnp.uint8)


def _blend_tc(image, outlier, mask):
    N, C, H, W = image.shape
    img_spec = pl.BlockSpec((1, C, H, W), lambda n: (n, 0, 0, 0))
    map_spec = pl.BlockSpec((1, H, W), lambda n: (n, 0, 0))
    return pl.pallas_call(
        _blend_kernel,
        grid=(N,),
        in_specs=[img_spec, img_spec, map_spec],
        out_specs=img_spec,
        out_shape=jax.ShapeDtypeStruct((N, C, H, W), jnp.uint8),
    )(image, outlier, mask)


_NC = 2
_NS = 16
_NW = _NC * _NS
_L = 16


def _label_sc_body(total, per_w, chunk, n_rounds,
                   label_hbm, mask_hbm, out_hbm,
                   l0, m0, o0, l1, m1, o1,
                   sem_in0, sem_in1, sem_out0, sem_out1):
    wid = lax.axis_index("s") * _NC + lax.axis_index("c")
    base = wid * per_w
    lbuf = (l0, l1)
    mbuf = (m0, m1)
    obuf = (o0, o1)
    sem_in = (sem_in0, sem_in1)
    sem_out = (sem_out0, sem_out1)

    def start_in(r, p):
        off = base + r * chunk
        cl = pltpu.make_async_copy(label_hbm.at[pl.ds(off, chunk)], lbuf[p],
                                   sem_in[p])
        cm = pltpu.make_async_copy(mask_hbm.at[pl.ds(off, chunk)], mbuf[p],
                                   sem_in[p])
        cl.start()
        cm.start()
        return (cl, cm)

    def start_out(r, p):
        off = base + r * chunk
        co = pltpu.make_async_copy(obuf[p], out_hbm.at[pl.ds(off, chunk)],
                                   sem_out[p])
        co.start()
        return co

    in_flight = {0: start_in(0, 0), 1: start_in(1, 1)}
    out_flight = {}

    hundred = jnp.full((_L,), 100, jnp.int32)

    unroll = 8
    step = _L * unroll

    for r in range(n_rounds):
        p = r % 2
        cl, cm = in_flight.pop(r)
        cl.wait()
        cm.wait()
        if r >= 2:
            out_flight.pop(r - 2).wait()

        lref, mref, oref = lbuf[p], mbuf[p], obuf[p]

        def body(i, _, lref=lref, mref=mref, oref=oref):
            b = i * step
            for u in range(unroll):
                sl = pl.ds(b + u * _L, _L)
                mv = mref[sl]
                lv = lref[sl]
                oref[sl] = jnp.where(mv != 0, hundred, lv)
            return 0

        lax.fori_loop(0, chunk // step, body, 0)

        out_flight[r] = start_out(r, p)
        if r + 2 < n_rounds:
            in_flight[r + 2] = start_in(r + 2, p)

    for r in sorted(out_flight):
        out_flight[r].wait()


def _label_sc(label_flat, mask_bits_flat):
    total = label_flat.shape[0]
    per_w = total // _NW
    chunk = 16384
    n_rounds = per_w // chunk

    mesh = plsc.VectorSubcoreMesh(core_axis_name="c", subcore_axis_name="s")
    kern = functools.partial(
        pl.kernel,
        mesh=mesh,
        out_type=jax.ShapeDtypeStruct((total,), jnp.int32),
        scratch_types=[
            pltpu.VMEM((chunk,), jnp.int32),
            pltpu.VMEM((chunk,), jnp.int32),
            pltpu.VMEM((chunk,), jnp.int32),
            pltpu.VMEM((chunk,), jnp.int32),
            pltpu.VMEM((chunk,), jnp.int32),
            pltpu.VMEM((chunk,), jnp.int32),
            pltpu.SemaphoreType.DMA,
            pltpu.SemaphoreType.DMA,
            pltpu.SemaphoreType.DMA,
            pltpu.SemaphoreType.DMA,
        ],
    )(functools.partial(_label_sc_body, total, per_w, chunk, n_rounds))
    return kern(label_flat, mask_bits_flat)


def kernel(image, label, outlier, mask):
    N, C, H, W = image.shape
    image_out = _blend_tc(image, outlier, mask)
    mask_bits = jax.lax.bitcast_convert_type(mask, jnp.int32).reshape(-1)
    label_out = _label_sc(label.reshape(-1), mask_bits).reshape(N, H, W)
    return (image_out, label_out)


# speedup vs baseline: 1.8422x; 1.8422x over previous
import functools

import jax
import jax.numpy as jnp
from jax import lax
from jax.experimental import pallas as pl
from jax.experimental.pallas import tpu as pltpu
from jax.experimental.pallas import tpu_sc as plsc

_ALPHA = 1.0


def _blend_kernel(image_ref, outlier_ref, mask_ref, image_out_ref):
    m = mask_ref[...]
    img = image_ref[...]
    out = outlier_ref[...]
    blended = img - _ALPHA * m[:, None] * img + _ALPHA * out
    image_out_ref[...] = blended.astype(jnp.uint8)


def _blend_tc(image, outlier, mask):
    N, C, H, W = image.shape
    img_spec = pl.BlockSpec((1, C, H, W), lambda n: (n, 0, 0, 0))
    map_spec = pl.BlockSpec((1, H, W), lambda n: (n, 0, 0))
    return pl.pallas_call(
        _blend_kernel,
        grid=(N,),
        in_specs=[img_spec, img_spec, map_spec],
        out_specs=img_spec,
        out_shape=jax.ShapeDtypeStruct((N, C, H, W), jnp.uint8),
    )(image, outlier, mask)


_NC = 2
_NS = 16
_NW = _NC * _NS
_L = 16


def _label_sc_body(N, H, W, rows_per_w, chunk_rows, n_rounds,
                   label_hbm, mask_hbm, out_hbm,
                   l0, m0, o0, l1, m1, o1,
                   sem_in0, sem_in1, sem_out0, sem_out1):
    wid = lax.axis_index("s") * _NC + lax.axis_index("c")
    workers_per_sample = H // rows_per_w
    n = wid // workers_per_sample
    row0 = (wid % workers_per_sample) * rows_per_w
    lbuf = (l0, l1)
    mbuf = (m0, m1)
    obuf = (o0, o1)
    sem_in = (sem_in0, sem_in1)
    sem_out = (sem_out0, sem_out1)

    def start_in(r, p):
        rs = pl.ds(row0 + r * chunk_rows, chunk_rows)
        cl = pltpu.make_async_copy(label_hbm.at[n, rs], lbuf[p], sem_in[p])
        cm = pltpu.make_async_copy(mask_hbm.at[n, rs], mbuf[p], sem_in[p])
        cl.start()
        cm.start()
        return (cl, cm)

    def start_out(r, p):
        rs = pl.ds(row0 + r * chunk_rows, chunk_rows)
        co = pltpu.make_async_copy(obuf[p], out_hbm.at[n, rs], sem_out[p])
        co.start()
        return co

    in_flight = {0: start_in(0, 0), 1: start_in(1, 1)}
    out_flight = {}

    hundred = jnp.full((_L,), 100, jnp.int32)
    nvec = W // _L

    for r in range(n_rounds):
        p = r % 2
        cl, cm = in_flight.pop(r)
        cl.wait()
        cm.wait()
        if r >= 2:
            out_flight.pop(r - 2).wait()

        lref, mref, oref = lbuf[p], mbuf[p], obuf[p]

        def body(i, _, lref=lref, mref=mref, oref=oref):
            for u in range(nvec):
                sl = pl.ds(u * _L, _L)
                mv = mref[i, sl]
                lv = lref[i, sl]
                oref[i, sl] = jnp.where(mv != 0.0, hundred, lv)
            return 0

        lax.fori_loop(0, chunk_rows, body, 0)

        out_flight[r] = start_out(r, p)
        if r + 2 < n_rounds:
            in_flight[r + 2] = start_in(r + 2, p)

    for r in sorted(out_flight):
        out_flight[r].wait()


def _label_sc(label, mask):
    N, H, W = label.shape
    rows_per_w = (N * H) // _NW
    chunk_rows = 32
    n_rounds = rows_per_w // chunk_rows

    mesh = plsc.VectorSubcoreMesh(core_axis_name="c", subcore_axis_name="s")
    kern = functools.partial(
        pl.kernel,
        mesh=mesh,
        out_type=jax.ShapeDtypeStruct((N, H, W), jnp.int32),
        scratch_types=[
            pltpu.VMEM((chunk_rows, W), jnp.int32),
            pltpu.VMEM((chunk_rows, W), jnp.float32),
            pltpu.VMEM((chunk_rows, W), jnp.int32),
            pltpu.VMEM((chunk_rows, W), jnp.int32),
            pltpu.VMEM((chunk_rows, W), jnp.float32),
            pltpu.VMEM((chunk_rows, W), jnp.int32),
            pltpu.SemaphoreType.DMA,
            pltpu.SemaphoreType.DMA,
            pltpu.SemaphoreType.DMA,
            pltpu.SemaphoreType.DMA,
        ],
    )(functools.partial(_label_sc_body, N, H, W, rows_per_w, chunk_rows,
                        n_rounds))
    return kern(label, mask)


def kernel(image, label, outlier, mask):
    N, C, H, W = image.shape
    image_out = _blend_tc(image, outlier, mask)
    label_out = _label_sc(label, mask)
    return (image_out, label_out)


# speedup vs baseline: 1.8463x; 1.0022x over previous
import functools

import jax
import jax.numpy as jnp
from jax import lax
from jax.experimental import pallas as pl
from jax.experimental.pallas import tpu as pltpu
from jax.experimental.pallas import tpu_sc as plsc

_ALPHA = 1.0


def _blend_kernel(image_ref, outlier_ref, mask_ref, image_out_ref):
    m = mask_ref[...]
    img = image_ref[...]
    out = outlier_ref[...]
    blended = img - _ALPHA * m[:, None] * img + _ALPHA * out
    image_out_ref[...] = blended.astype(jnp.uint8)


def _blend_tc(image, outlier, mask):
    N, C, H, W = image.shape
    img_spec = pl.BlockSpec((1, C, H, W), lambda n: (n, 0, 0, 0))
    map_spec = pl.BlockSpec((1, H, W), lambda n: (n, 0, 0))
    return pl.pallas_call(
        _blend_kernel,
        grid=(N,),
        in_specs=[img_spec, img_spec, map_spec],
        out_specs=img_spec,
        out_shape=jax.ShapeDtypeStruct((N, C, H, W), jnp.uint8),
    )(image, outlier, mask)


_NC = 2
_NS = 16
_NW = _NC * _NS
_L = 16


def _label_sc_body(N, H, W, rows_per_w, chunk_rows, n_rounds,
                   label_hbm, mask_hbm, out_hbm,
                   l0, m0, o0, l1, m1, o1,
                   sem_in0, sem_in1, sem_out0, sem_out1):
    wid = lax.axis_index("s") * _NC + lax.axis_index("c")
    workers_per_sample = H // rows_per_w
    n = wid // workers_per_sample
    row0 = (wid % workers_per_sample) * rows_per_w
    lbuf = (l0, l1)
    mbuf = (m0, m1)
    obuf = (o0, o1)
    sem_in = (sem_in0, sem_in1)
    sem_out = (sem_out0, sem_out1)

    def start_in(r, p):
        rs = pl.ds(row0 + r * chunk_rows, chunk_rows)
        cl = pltpu.make_async_copy(label_hbm.at[n, rs], lbuf[p], sem_in[p])
        cm = pltpu.make_async_copy(mask_hbm.at[n, rs], mbuf[p], sem_in[p])
        cl.start()
        cm.start()
        return (cl, cm)

    def start_out(r, p):
        rs = pl.ds(row0 + r * chunk_rows, chunk_rows)
        co = pltpu.make_async_copy(obuf[p], out_hbm.at[n, rs], sem_out[p])
        co.start()
        return co

    in_flight = {0: start_in(0, 0), 1: start_in(1, 1)}
    out_flight = {}

    hundred = jnp.full((_L,), 100, jnp.int32)
    nvec = W // _L

    for r in range(n_rounds):
        p = r % 2
        cl, cm = in_flight.pop(r)
        cl.wait()
        cm.wait()
        if r >= 2:
            out_flight.pop(r - 2).wait()

        lref, mref, oref = lbuf[p], mbuf[p], obuf[p]

        def body(i, _, lref=lref, mref=mref, oref=oref):
            for u in range(nvec):
                sl = pl.ds(u * _L, _L)
                mv = mref[i, sl]
                lv = lref[i, sl]
                oref[i, sl] = jnp.where(mv != 0.0, hundred, lv)
            return 0

        lax.fori_loop(0, chunk_rows, body, 0)

        out_flight[r] = start_out(r, p)
        if r + 2 < n_rounds:
            in_flight[r + 2] = start_in(r + 2, p)

    for r in sorted(out_flight):
        out_flight[r].wait()


def _label_sc(label, mask):
    N, H, W = label.shape
    rows_per_w = (N * H) // _NW
    chunk_rows = 32
    n_rounds = rows_per_w // chunk_rows

    mesh = plsc.VectorSubcoreMesh(core_axis_name="c", subcore_axis_name="s")
    kern = functools.partial(
        pl.kernel,
        mesh=mesh,
        out_type=jax.ShapeDtypeStruct((N, H, W), jnp.int32),
        scratch_types=[
            pltpu.VMEM((chunk_rows, W), jnp.int32),
            pltpu.VMEM((chunk_rows, W), jnp.float32),
            pltpu.VMEM((chunk_rows, W), jnp.int32),
            pltpu.VMEM((chunk_rows, W), jnp.int32),
            pltpu.VMEM((chunk_rows, W), jnp.float32),
            pltpu.VMEM((chunk_rows, W), jnp.int32),
            pltpu.SemaphoreType.DMA,
            pltpu.SemaphoreType.DMA,
            pltpu.SemaphoreType.DMA,
            pltpu.SemaphoreType.DMA,
        ],
    )(functools.partial(_label_sc_body, N, H, W, rows_per_w, chunk_rows,
                        n_rounds))
    return kern(label, mask)


def kernel(image, label, outlier, mask):
    N, C, H, W = image.shape
    label_out = _label_sc(label, mask)
    image_out = _blend_tc(image, outlier, mask)
    return (image_out, label_out)


# speedup vs baseline: 2.3263x; 1.2600x over previous
import jax
import jax.numpy as jnp
from jax.experimental import pallas as pl

_ALPHA = 1.0


def _blend_kernel(image_ref, outlier_ref, mask_ref, image_out_ref):
    m = mask_ref[...]
    img = image_ref[...]
    out = outlier_ref[...]
    blended = img - _ALPHA * m[:, None] * img + _ALPHA * out
    image_out_ref[...] = blended.astype(jnp.uint8)


def _blend_tc(image, outlier, mask):
    N, C, H, W = image.shape
    img_spec = pl.BlockSpec((1, C, H, W), lambda n: (n, 0, 0, 0))
    map_spec = pl.BlockSpec((1, H, W), lambda n: (n, 0, 0))
    return pl.pallas_call(
        _blend_kernel,
        grid=(N,),
        in_specs=[img_spec, img_spec, map_spec],
        out_specs=img_spec,
        out_shape=jax.ShapeDtypeStruct((N, C, H, W), jnp.uint8),
    )(image, outlier, mask)


def _label_kernel(label_ref, mask_ref, label_out_ref):
    m = mask_ref[...]
    label_out_ref[...] = jnp.where(m != 0.0, jnp.int32(100), label_ref[...])


def _label_tc(label, mask):
    N, H, W = label.shape
    spec = pl.BlockSpec((2, H, W), lambda n: (n, 0, 0))
    return pl.pallas_call(
        _label_kernel,
        grid=(N // 2,),
        in_specs=[spec, spec],
        out_specs=spec,
        out_shape=jax.ShapeDtypeStruct((N, H, W), jnp.int32),
    )(label, mask)


def kernel(image, label, outlier, mask):
    image_out = _blend_tc(image, outlier, mask)
    label_out = _label_tc(label, mask)
    return (image_out, label_out)


# speedup vs baseline: 2.6652x; 1.1457x over previous
import jax
import jax.numpy as jnp
from jax.experimental import pallas as pl

_ALPHA = 1.0


def _fused_kernel(image_ref, label_ref, outlier_ref, mask_ref,
                  image_out_ref, label_out_ref):
    m = mask_ref[...]
    img = image_ref[...]
    out = outlier_ref[...]
    blended = img - _ALPHA * m[:, None] * img + _ALPHA * out
    image_out_ref[...] = blended.astype(jnp.uint8)
    lbl = label_ref[...]
    label_out_ref[...] = jnp.where(m != 0.0, jnp.int32(100), lbl)


def kernel(image, label, outlier, mask):
    N, C, H, W = image.shape
    img_spec = pl.BlockSpec((1, C, H, W), lambda n: (n, 0, 0, 0))
    map_spec = pl.BlockSpec((1, H, W), lambda n: (n, 0, 0))

    image_out, label_out = pl.pallas_call(
        _fused_kernel,
        grid=(N,),
        in_specs=[img_spec, map_spec, img_spec, map_spec],
        out_specs=[img_spec, map_spec],
        out_shape=[
            jax.ShapeDtypeStruct((N, C, H, W), jnp.uint8),
            jax.ShapeDtypeStruct((N, H, W), label.dtype),
        ],
    )(image, label, outlier, mask)
    return (image_out, label_out)
